# baseline (speedup 1.0000x reference)
import functools

import jax
import jax.numpy as jnp
from jax import lax
from jax.experimental import pallas as pl
from jax.experimental.pallas import tpu as pltpu
from jax.experimental.pallas import tpu_sc as plsc

NC = 2
NS = 16
IB = 1024
DR = 128
CAP = 2 * IB + DR


def _make_sc_call(R, D, C, per_tile, N):
    n_chunks = R // C
    n_chunks_per_core = n_chunks // NC
    mesh = plsc.VectorSubcoreMesh(
        core_axis_name="c", subcore_axis_name="s",
        num_cores=NC, num_subcores=NS)
    acc_rows = -(-(C + 1) // (8 * NS)) * 8 * NS
    zrows = acc_rows // NS
    wrows = (C // NS) // 8 * 8
    wrem = C - wrows * NS
    nblocks = per_tile // IB
    MAXD = 2 * IB // DR

    @functools.partial(
        pl.kernel,
        out_type=jax.ShapeDtypeStruct((R, D), jnp.float32),
        mesh=mesh,
        compiler_params=pltpu.CompilerParams(needs_layout_passes=False),
        scratch_types=[
            pltpu.VMEM_SHARED((acc_rows, 128), jnp.float32),
            pltpu.VMEM((IB,), jnp.int32),
            pltpu.VMEM((IB,), jnp.int32),
            pltpu.VMEM((IB,), jnp.int32),
            pltpu.VMEM((IB,), jnp.int32),
            pltpu.VMEM((IB,), jnp.int32),
            pltpu.VMEM((IB,), jnp.int32),
            pltpu.VMEM((CAP,), jnp.int32),
            pltpu.VMEM((CAP,), jnp.int32),
            pltpu.VMEM((DR,), jnp.int32),
            pltpu.VMEM((DR,), jnp.int32),
            pltpu.VMEM((DR, 128), jnp.float32),
            pltpu.VMEM((DR, 128), jnp.float32),
            pltpu.SemaphoreType.DMA,
            pltpu.SemaphoreType.DMA,
            pltpu.SemaphoreType.DMA,
            pltpu.SemaphoreType.DMA,
        ],
    )
    def sc_call(xf_h, bi_h, si_h, di_h, z_h, out_h,
                acc, biA, siA, diA, biB, siB, diB, cg, cs,
                sx0, sx1, st0, st1, semA, semB, sem0, sem1):
        c = lax.axis_index("c")
        s = lax.axis_index("s")
        sts = (st0, st1)
        sems = (sem0, sem1)
        sxs = (sx0, sx1)

        def issue_idx(blk, bufs, sem):
            base = s * per_tile + blk * IB
            pltpu.async_copy(bi_h.at[pl.ds(base, IB)], bufs[0], sem)
            pltpu.async_copy(si_h.at[pl.ds(base, IB)], bufs[1], sem)
            pltpu.async_copy(di_h.at[pl.ds(base, IB)], bufs[2], sem)

        def wait_idx(bufs, sem):
            for b in bufs:
                pltpu.make_async_copy(bi_h.at[pl.ds(0, IB)], b, sem).wait()

        def process(blk, bufs, sem, nxt_blk, nxt_bufs, nxt_sem, guard, lo, hi):
            if guard:
                @pl.when(nxt_blk < nblocks)
                def _():
                    issue_idx(nxt_blk, nxt_bufs, nxt_sem)
            else:
                issue_idx(nxt_blk, nxt_bufs, nxt_sem)
            wait_idx(bufs, sem)
            biX, siX, diX = bufs

            def comp(t, cnt):
                for u in range(4):
                    sl = pl.ds((t * 4 + u) * 16, 16)
                    vb = biX[sl]
                    ls = vb * N + siX[sl]
                    ld = vb * N + diX[sl]
                    m0 = (ld >= lo) & (ld < hi)
                    m0i = jnp.where(m0, 1, 0)
                    pos0 = cnt + plsc.cumsum(m0i) - m0i
                    plsc.store_scatter(cg, [pos0], ls, mask=m0)
                    plsc.store_scatter(cs, [pos0], ld - lo, mask=m0)
                    cnt = cnt + plsc.all_reduce_population_count(m0)
                    m1 = (ls >= lo) & (ls < hi)
                    m1i = jnp.where(m1, 1, 0)
                    pos1 = cnt + plsc.cumsum(m1i) - m1i
                    plsc.store_scatter(cg, [pos1], ld, mask=m1)
                    plsc.store_scatter(cs, [pos1], ls - lo, mask=m1)
                    cnt = cnt + plsc.all_reduce_population_count(m1)
                return cnt

            cntv = lax.fori_loop(0, IB // 64, comp,
                                 jnp.zeros((16,), jnp.int32))
            cnt = cntv[0]
            for t in range(DR // 16):
                sl = pl.ds(cnt + t * 16, 16)
                cg[sl] = jnp.zeros((16,), jnp.int32)
                cs[sl] = jnp.full((16,), C, jnp.int32)
            ndrain = (cnt + DR - 1) // DR

            @pl.when(ndrain > 0)
            def _():
                pltpu.async_copy(xf_h.at[cg.at[pl.ds(0, DR)]], st0, sem0)
            for d in range(MAXD):
                @pl.when(d < ndrain)
                def _(d=d):
                    pltpu.make_async_copy(xf_h.at[pl.ds(0, DR)],
                                          sts[d % 2], sems[d % 2]).wait()
                    if d + 1 < MAXD:
                        @pl.when(d + 1 < ndrain)
                        def _():
                            pltpu.async_copy(
                                xf_h.at[cg.at[pl.ds((d + 1) * DR, DR)]],
                                sts[(d + 1) % 2], sems[(d + 1) % 2])
                    sx = sxs[d % 2]
                    for t in range(DR // 16):
                        sx[pl.ds(t * 16, 16)] = cs[pl.ds(d * DR + t * 16, 16)]
                    pltpu.sync_copy(sts[d % 2], acc.at[sx], add=True)

        def chunk_body(p, carry):
            lo = (n_chunks_per_core * c + p) * C
            hi = lo + C
            pltpu.sync_copy(z_h.at[pl.ds(s * zrows, zrows)],
                            acc.at[pl.ds(s * zrows, zrows)])
            plsc.subcore_barrier()
            issue_idx(0, (biA, siA, diA), semA)

            def pair_body(k, carry2):
                b0 = 2 * k
                process(b0, (biA, siA, diA), semA,
                        b0 + 1, (biB, siB, diB), semB, False, lo, hi)
                process(b0 + 1, (biB, siB, diB), semB,
                        b0 + 2, (biA, siA, diA), semA, True, lo, hi)
                return carry2

            lax.fori_loop(0, nblocks // 2, pair_body, 0)
            plsc.subcore_barrier()
            pltpu.sync_copy(acc.at[pl.ds(s * wrows, wrows)],
                            out_h.at[pl.ds(lo + s * wrows, wrows)])
            if wrem:
                @pl.when(s == 0)
                def _():
                    pltpu.sync_copy(acc.at[pl.ds(NS * wrows, wrem)],
                                    out_h.at[pl.ds(lo + NS * wrows, wrem)])
            plsc.subcore_barrier()
            return carry

        lax.fori_loop(0, n_chunks_per_core, chunk_body, 0)

    return sc_call


def kernel(x, batch_idx, src, dst):
    B, N, D = x.shape
    R = B * N
    C = R // 4
    E = batch_idx.shape[0]
    per_tile = -(-E // (NS * 2 * IB)) * 2 * IB
    pad = per_tile * NS - E

    bi = jnp.concatenate([batch_idx.astype(jnp.int32),
                          jnp.full((pad,), -1, jnp.int32)])
    si = jnp.concatenate([src.astype(jnp.int32), jnp.zeros((pad,), jnp.int32)])
    di = jnp.concatenate([dst.astype(jnp.int32), jnp.zeros((pad,), jnp.int32)])
    xf = x.reshape(R, D)
    acc_rows = -(-(C + 1) // (8 * NS)) * 8 * NS
    z = jnp.zeros((acc_rows, D), jnp.float32)

    sc_call = _make_sc_call(R, D, C, per_tile, N)
    out_sum = sc_call(xf, bi, si, di, z)
    return jnp.concatenate([out_sum.reshape(B, N, D), x], axis=2)


# speedup vs baseline: 1.0005x; 1.0005x over previous
import functools

import jax
import jax.numpy as jnp
from jax import lax
from jax.experimental import pallas as pl
from jax.experimental.pallas import tpu as pltpu
from jax.experimental.pallas import tpu_sc as plsc

NC = 2
NS = 16
IB = 1024
DR = 128
CAP = 2 * IB + DR


def _make_sc_call(R, D, C, per_tile, N):
    n_chunks_per_core = (R // C) // NC
    mesh = plsc.VectorSubcoreMesh(
        core_axis_name="c", subcore_axis_name="s",
        num_cores=NC, num_subcores=NS)
    acc_rows = -(-(C + 1) // (8 * NS)) * 8 * NS
    zrows = acc_rows // NS
    wrows = (C // NS) // 8 * 8
    assert acc_rows % (8 * NS) == 0
    wrem = C - wrows * NS
    nblocks = per_tile // IB
    MAXD = -(-2 * IB // DR)
    DRR = DR // 128

    @functools.partial(
        pl.kernel,
        out_type=jax.ShapeDtypeStruct((R, D), jnp.float32),
        mesh=mesh,
        compiler_params=pltpu.CompilerParams(needs_layout_passes=False),
        scratch_types=[
            pltpu.VMEM_SHARED((acc_rows, 128), jnp.float32),
            pltpu.VMEM((IB,), jnp.int32),
            pltpu.VMEM((IB,), jnp.int32),
            pltpu.VMEM((IB,), jnp.int32),
            pltpu.VMEM((IB,), jnp.int32),
            pltpu.VMEM((IB,), jnp.int32),
            pltpu.VMEM((IB,), jnp.int32),
            pltpu.VMEM((CAP,), jnp.int32),
            pltpu.VMEM((CAP,), jnp.int32),
            pltpu.VMEM((DR,), jnp.int32),
            pltpu.VMEM((DR,), jnp.int32),
            pltpu.VMEM((DR,), jnp.int32),
            pltpu.VMEM((DR,), jnp.int32),
            pltpu.VMEM((DR, 128), jnp.float32),
            pltpu.VMEM((DR, 128), jnp.float32),
            pltpu.SemaphoreType.DMA,
            pltpu.SemaphoreType.DMA,
            pltpu.SemaphoreType.DMA,
            pltpu.SemaphoreType.DMA,
            pltpu.SemaphoreType.DMA,
            pltpu.SemaphoreType.DMA,
        ],
    )
    def sc_call(xf_h, bi_h, si_h, di_h, z_h, out_h,
                acc, biA, siA, diA, biB, siB, diB, cg, cs,
                gx0, gx1, sx0, sx1, st0, st1,
                semA, semB, semG0, semG1, semS0, semS1):
        c = lax.axis_index("c")
        s = lax.axis_index("s")
        sts = (st0, st1)
        gxs = (gx0, gx1)
        sxs = (sx0, sx1)
        semG = (semG0, semG1)
        semS = (semS0, semS1)

        def issue_idx(blk, bufs, sem):
            base = s * per_tile + blk * IB
            pltpu.async_copy(bi_h.at[pl.ds(base, IB)], bufs[0], sem)
            pltpu.async_copy(si_h.at[pl.ds(base, IB)], bufs[1], sem)
            pltpu.async_copy(di_h.at[pl.ds(base, IB)], bufs[2], sem)

        def wait_idx(bufs, sem):
            for b in bufs:
                pltpu.make_async_copy(bi_h.at[pl.ds(0, IB)], b, sem).wait()

        def issue_gather(d, k):
            gx = gxs[k]
            for t in range(DR // 16):
                gx[pl.ds(t * 16, 16)] = cg[pl.ds(d * DR + t * 16, 16)]
            pltpu.async_copy(xf_h.at[gx], sts[k], semG[k])

        def wait_gather(k):
            pltpu.make_async_copy(xf_h.at[gxs[k]],
                                  sts[k], semG[k]).wait()

        def do_scatter(d, k):
            sx = sxs[k]
            for t in range(DR // 16):
                sx[pl.ds(t * 16, 16)] = cs[pl.ds(d * DR + t * 16, 16)]
            pltpu.sync_copy(sts[k], acc.at[sx], add=True)

        def append(cnt, vals_g, vals_s, m):
            mi = jnp.where(m, 1, 0)
            pos = cnt + plsc.cumsum(mi) - mi
            plsc.store_scatter(cg, [pos], vals_g, mask=m)
            plsc.store_scatter(cs, [pos], vals_s, mask=m)
            return cnt + plsc.all_reduce_population_count(m)

        def process(bufs, sem, nxt_blk, nxt_bufs, nxt_sem, guard, lo, hi):
            if guard:
                @pl.when(nxt_blk < nblocks)
                def _():
                    issue_idx(nxt_blk, nxt_bufs, nxt_sem)
            else:
                issue_idx(nxt_blk, nxt_bufs, nxt_sem)
            wait_idx(bufs, sem)
            biX, siX, diX = bufs

            def comp(t, cnt):
                for u in range(4):
                    sl = pl.ds((t * 4 + u) * 16, 16)
                    vb = biX[sl]
                    ls = vb * N + siX[sl]
                    ld = vb * N + diX[sl]
                    cnt = append(cnt, ls, ld - lo, (ld >= lo) & (ld < hi))
                    cnt = append(cnt, ld, ls - lo, (ls >= lo) & (ls < hi))
                return cnt

            cntv = lax.fori_loop(0, IB // 64, comp,
                                 jnp.zeros((16,), jnp.int32))
            cnt = cntv[0]
            iota = lax.iota(jnp.int32, 16)
            zg = jnp.zeros((16,), jnp.int32)
            zs = jnp.full((16,), C, jnp.int32)
            for t in range(DR // 16):
                pos = cnt + t * 16 + iota
                plsc.store_scatter(cg, [pos], zg)
                plsc.store_scatter(cs, [pos], zs)
            nd = (cnt + DR - 1) // DR

            @pl.when(nd > 0)
            def _():
                issue_gather(0, 0)
            for d in range(MAXD):
                k = d % 2

                @pl.when(d < nd)
                def _(d=d, k=k):
                    if d + 1 < MAXD:
                        @pl.when(d + 1 < nd)
                        def _():
                            issue_gather(d + 1, 1 - k)
                    wait_gather(k)
                    do_scatter(d, k)

        def chunk_body(p, carry):
            lo = (n_chunks_per_core * c + p) * C
            hi = lo + C
            pltpu.sync_copy(z_h.at[pl.ds(s * zrows, zrows)],
                            acc.at[pl.ds(s * zrows, zrows)])
            plsc.subcore_barrier()
            issue_idx(0, (biA, siA, diA), semA)

            def pair_body(q, carry2):
                b0 = 2 * q
                process((biA, siA, diA), semA,
                        b0 + 1, (biB, siB, diB), semB, False, lo, hi)
                process((biB, siB, diB), semB,
                        b0 + 2, (biA, siA, diA), semA, True, lo, hi)
                return carry2

            lax.fori_loop(0, nblocks // 2, pair_body, 0)
            plsc.subcore_barrier()
            pltpu.sync_copy(acc.at[pl.ds(s * wrows, wrows)],
                            out_h.at[pl.ds(lo + s * wrows, wrows)])
            if wrem:
                @pl.when(s == 0)
                def _():
                    pltpu.sync_copy(acc.at[pl.ds(NS * wrows, wrem)],
                                    out_h.at[pl.ds(lo + NS * wrows, wrem)])
            plsc.subcore_barrier()
            return carry

        lax.fori_loop(0, n_chunks_per_core, chunk_body, 0)

    return sc_call


def kernel(x, batch_idx, src, dst):
    B, N, D = x.shape
    R = B * N
    C = R // 4
    E = batch_idx.shape[0]
    per_tile = -(-E // (NS * 2 * IB)) * 2 * IB
    pad = per_tile * NS - E

    bi = jnp.concatenate([batch_idx.astype(jnp.int32),
                          jnp.full((pad,), -1, jnp.int32)])
    si = jnp.concatenate([src.astype(jnp.int32), jnp.zeros((pad,), jnp.int32)])
    di = jnp.concatenate([dst.astype(jnp.int32), jnp.zeros((pad,), jnp.int32)])
    xf = x.reshape(R, D)
    acc_rows = -(-(C + 1) // (8 * NS)) * 8 * NS
    z = jnp.zeros((acc_rows, D), jnp.float32)

    sc_call = _make_sc_call(R, D, C, per_tile, N)
    out_sum = sc_call(xf, bi, si, di, z)
    return jnp.concatenate([out_sum.reshape(B, N, D), x], axis=2)


# speedup vs baseline: 1.1987x; 1.1981x over previous
import functools

import jax
import jax.numpy as jnp
from jax import lax
from jax.experimental import pallas as pl
from jax.experimental.pallas import tpu as pltpu
from jax.experimental.pallas import tpu_sc as plsc

NC = 2
NS = 16
IB = 512
BR = 128
NST = 2


def _make_sc_call(R, D, C, per_tile, N):
    n_chunks_per_core = (R // C) // NC
    mesh = plsc.VectorSubcoreMesh(
        core_axis_name="c", subcore_axis_name="s",
        num_cores=NC, num_subcores=NS)
    acc_rows = -(-(C + 1) // (8 * NS)) * 8 * NS
    zrows = acc_rows // NS
    wrows = (C // NS) // 8 * 8
    wrem = C - wrows * NS
    nblocks = per_tile // IB
    UPB = (IB // BR) * 2
    T = nblocks * UPB
    assert UPB % NST == 0 and T % NST == 0

    @functools.partial(
        pl.kernel,
        out_type=jax.ShapeDtypeStruct((R, D), jnp.float32),
        mesh=mesh,
        scratch_types=[
            pltpu.VMEM_SHARED((acc_rows, 128), jnp.float32),
            pltpu.VMEM((IB,), jnp.int32),
            pltpu.VMEM((IB,), jnp.int32),
            pltpu.VMEM((IB,), jnp.int32),
            pltpu.VMEM((IB,), jnp.int32),
            pltpu.VMEM((IB,), jnp.int32),
            pltpu.VMEM((IB,), jnp.int32),
            [pltpu.VMEM((BR,), jnp.int32) for _ in range(NST)],
            [pltpu.VMEM((BR,), jnp.int32) for _ in range(NST)],
            [pltpu.VMEM((BR, 128), jnp.float32) for _ in range(NST)],
            pltpu.SemaphoreType.DMA,
            pltpu.SemaphoreType.DMA,
            [pltpu.SemaphoreType.DMA for _ in range(NST)],
            [pltpu.SemaphoreType.DMA for _ in range(NST)],
        ],
    )
    def sc_call(xf_h, bi_h, si_h, di_h, z_h, out_h,
                acc, biA, siA, diA, biB, siB, diB, gxs, sxs, sts,
                semA, semB, semG, semS):
        c = lax.axis_index("c")
        s = lax.axis_index("s")

        def issue_idx(blk, bufs, sem):
            base = s * per_tile + blk * IB
            pltpu.async_copy(bi_h.at[pl.ds(base, IB)], bufs[0], sem)
            pltpu.async_copy(si_h.at[pl.ds(base, IB)], bufs[1], sem)
            pltpu.async_copy(di_h.at[pl.ds(base, IB)], bufs[2], sem)

        def wait_idx(bufs, sem):
            for b in bufs:
                pltpu.make_async_copy(bi_h.at[pl.ds(0, IB)], b, sem).wait()

        def issue_gather(k):
            pltpu.async_copy(xf_h.at[gxs[k]], sts[k], semG[k])

        def wait_gather(k):
            pltpu.make_async_copy(xf_h.at[gxs[k]], sts[k], semG[k]).wait()

        def issue_scatter(k):
            pltpu.async_copy(sts[k], acc.at[sxs[k]], semS[k], add=True)

        def wait_scatter(k):
            pltpu.make_async_copy(sts[k], acc.at[sxs[k]], semS[k]).wait()

        def units(blk, bufs, lo):
            biX, siX, diX = bufs
            for u in range(UPB):
                k = u % NST
                bat = u // 2
                if u >= NST:
                    wait_scatter(k)
                else:
                    @pl.when(blk > 0)
                    def _(k=k):
                        wait_scatter(k)
                for j in range(BR // 16):
                    sl = pl.ds(bat * BR + j * 16, 16)
                    vb = biX[sl]
                    if u % 2 == 0:
                        g = vb * N + siX[sl]
                        t = vb * N + diX[sl]
                    else:
                        g = vb * N + diX[sl]
                        t = vb * N + siX[sl]
                    gxs[k][pl.ds(j * 16, 16)] = jnp.maximum(g, 0)
                    tl = t - lo
                    sxs[k][pl.ds(j * 16, 16)] = jnp.where(
                        (tl < 0) | (tl >= C), C, tl)
                issue_gather(k)
                k2 = (u - 1) % NST
                if u >= 1:
                    wait_gather(k2)
                    issue_scatter(k2)
                else:
                    @pl.when(blk > 0)
                    def _(k2=k2):
                        wait_gather(k2)
                        issue_scatter(k2)

        def chunk_body(p, carry):
            lo = (n_chunks_per_core * c + p) * C
            pltpu.sync_copy(z_h.at[pl.ds(s * zrows, zrows)],
                            acc.at[pl.ds(s * zrows, zrows)])
            plsc.subcore_barrier()
            issue_idx(0, (biA, siA, diA), semA)

            def pair_body(q, carry2):
                b0 = 2 * q
                issue_idx(b0 + 1, (biB, siB, diB), semB)
                wait_idx((biA, siA, diA), semA)
                units(b0, (biA, siA, diA), lo)

                @pl.when(b0 + 2 < nblocks)
                def _():
                    issue_idx(b0 + 2, (biA, siA, diA), semA)
                wait_idx((biB, siB, diB), semB)
                units(b0 + 1, (biB, siB, diB), lo)
                return carry2

            lax.fori_loop(0, nblocks // 2, pair_body, 0)
            wait_gather((T - 1) % NST)
            issue_scatter((T - 1) % NST)
            wait_scatter((T - 2) % NST)
            wait_scatter((T - 1) % NST)
            plsc.subcore_barrier()
            pltpu.sync_copy(acc.at[pl.ds(s * wrows, wrows)],
                            out_h.at[pl.ds(lo + s * wrows, wrows)])
            if wrem:
                @pl.when(s == 0)
                def _():
                    pltpu.sync_copy(acc.at[pl.ds(NS * wrows, wrem)],
                                    out_h.at[pl.ds(lo + NS * wrows, wrem)])
            plsc.subcore_barrier()
            return carry

        lax.fori_loop(0, n_chunks_per_core, chunk_body, 0)

    return sc_call


def kernel(x, batch_idx, src, dst):
    B, N, D = x.shape
    R = B * N
    C = R // 4
    E = batch_idx.shape[0]
    per_tile = -(-E // (NS * 2 * IB)) * 2 * IB
    pad = per_tile * NS - E

    bi = jnp.concatenate([batch_idx.astype(jnp.int32),
                          jnp.full((pad,), -1, jnp.int32)])
    si = jnp.concatenate([src.astype(jnp.int32), jnp.zeros((pad,), jnp.int32)])
    di = jnp.concatenate([dst.astype(jnp.int32), jnp.zeros((pad,), jnp.int32)])
    xf = x.reshape(R, D)
    acc_rows = -(-(C + 1) // (8 * NS)) * 8 * NS
    z = jnp.zeros((acc_rows, D), jnp.float32)

    sc_call = _make_sc_call(R, D, C, per_tile, N)
    out_sum = sc_call(xf, bi, si, di, z)
    return jnp.concatenate([out_sum.reshape(B, N, D), x], axis=2)


# speedup vs baseline: 1.1990x; 1.0002x over previous
import functools

import jax
import jax.numpy as jnp
from jax import lax
from jax.experimental import pallas as pl
from jax.experimental.pallas import tpu as pltpu
from jax.experimental.pallas import tpu_sc as plsc

NC = 2
NS = 16
IB = 512
BR = 128
NST = 2


def _make_sc_call(R, D, C, per_tile, N):
    n_chunks_per_core = (R // C) // NC
    mesh = plsc.VectorSubcoreMesh(
        core_axis_name="c", subcore_axis_name="s",
        num_cores=NC, num_subcores=NS)
    acc_rows = -(-(C + 1) // (8 * NS)) * 8 * NS
    zrows = acc_rows // NS
    wrows = (C // NS) // 8 * 8
    wrem = C - wrows * NS
    nblocks = per_tile // IB
    UPB = (IB // BR) * 2
    T = nblocks * UPB
    assert UPB % NST == 0 and T % NST == 0

    @functools.partial(
        pl.kernel,
        out_type=jax.ShapeDtypeStruct((R, D), jnp.float32),
        mesh=mesh,
        scratch_types=[
            pltpu.VMEM_SHARED((acc_rows, 128), jnp.float32),
            pltpu.VMEM((IB,), jnp.int32),
            pltpu.VMEM((IB,), jnp.int32),
            pltpu.VMEM((IB,), jnp.int32),
            pltpu.VMEM((IB,), jnp.int32),
            pltpu.VMEM((IB,), jnp.int32),
            pltpu.VMEM((IB,), jnp.int32),
            [pltpu.VMEM((BR,), jnp.int32) for _ in range(NST)],
            [pltpu.VMEM((BR,), jnp.int32) for _ in range(NST)],
            [pltpu.VMEM((BR, 128), jnp.float32) for _ in range(NST)],
            pltpu.SemaphoreType.DMA,
            pltpu.SemaphoreType.DMA,
            [pltpu.SemaphoreType.DMA for _ in range(NST)],
        ],
    )
    def sc_call(xf_h, bi_h, si_h, di_h, z_h, out_h,
                acc, biA, siA, diA, biB, siB, diB, gxs, sxs, sts,
                semA, semB, semG):
        c = lax.axis_index("c")
        s = lax.axis_index("s")

        def issue_idx(blk, bufs, sem):
            base = s * per_tile + blk * IB
            pltpu.async_copy(bi_h.at[pl.ds(base, IB)], bufs[0], sem)
            pltpu.async_copy(si_h.at[pl.ds(base, IB)], bufs[1], sem)
            pltpu.async_copy(di_h.at[pl.ds(base, IB)], bufs[2], sem)

        def wait_idx(bufs, sem):
            for b in bufs:
                pltpu.make_async_copy(bi_h.at[pl.ds(0, IB)], b, sem).wait()

        def issue_gather(k):
            pltpu.async_copy(xf_h.at[gxs[k]], sts[k], semG[k])

        def wait_gather(k):
            pltpu.make_async_copy(xf_h.at[gxs[k]], sts[k], semG[k]).wait()

        def units(blk, bufs, lo):
            biX, siX, diX = bufs
            for u in range(UPB):
                k = u % NST
                bat = u // 2
                for j in range(BR // 16):
                    sl = pl.ds(bat * BR + j * 16, 16)
                    vb = biX[sl]
                    if u % 2 == 0:
                        g = vb * N + siX[sl]
                        t = vb * N + diX[sl]
                    else:
                        g = vb * N + diX[sl]
                        t = vb * N + siX[sl]
                    gxs[k][pl.ds(j * 16, 16)] = jnp.maximum(g, 0)
                    tl = t - lo
                    sxs[k][pl.ds(j * 16, 16)] = jnp.where(
                        (tl < 0) | (tl >= C), C, tl)
                issue_gather(k)
                k2 = 1 - k
                if u >= 1:
                    wait_gather(k2)
                    pltpu.sync_copy(sts[k2], acc.at[sxs[k2]], add=True)
                else:
                    @pl.when(blk > 0)
                    def _():
                        wait_gather(k2)
                        pltpu.sync_copy(sts[k2], acc.at[sxs[k2]], add=True)

        def chunk_body(p, carry):
            lo = (n_chunks_per_core * c + p) * C
            pltpu.sync_copy(z_h.at[pl.ds(s * zrows, zrows)],
                            acc.at[pl.ds(s * zrows, zrows)])
            plsc.subcore_barrier()
            issue_idx(0, (biA, siA, diA), semA)

            def pair_body(q, carry2):
                b0 = 2 * q
                issue_idx(b0 + 1, (biB, siB, diB), semB)
                wait_idx((biA, siA, diA), semA)
                units(b0, (biA, siA, diA), lo)

                @pl.when(b0 + 2 < nblocks)
                def _():
                    issue_idx(b0 + 2, (biA, siA, diA), semA)
                wait_idx((biB, siB, diB), semB)
                units(b0 + 1, (biB, siB, diB), lo)
                return carry2

            lax.fori_loop(0, nblocks // 2, pair_body, 0)
            wait_gather(1)
            pltpu.sync_copy(sts[1], acc.at[sxs[1]], add=True)
            plsc.subcore_barrier()
            pltpu.sync_copy(acc.at[pl.ds(s * wrows, wrows)],
                            out_h.at[pl.ds(lo + s * wrows, wrows)])
            if wrem:
                @pl.when(s == 0)
                def _():
                    pltpu.sync_copy(acc.at[pl.ds(NS * wrows, wrem)],
                                    out_h.at[pl.ds(lo + NS * wrows, wrem)])
            plsc.subcore_barrier()
            return carry

        lax.fori_loop(0, n_chunks_per_core, chunk_body, 0)

    return sc_call


def kernel(x, batch_idx, src, dst):
    B, N, D = x.shape
    R = B * N
    C = R // 4
    E = batch_idx.shape[0]
    per_tile = -(-E // (NS * 2 * IB)) * 2 * IB
    pad = per_tile * NS - E

    bi = jnp.concatenate([batch_idx.astype(jnp.int32),
                          jnp.full((pad,), -1, jnp.int32)])
    si = jnp.concatenate([src.astype(jnp.int32), jnp.zeros((pad,), jnp.int32)])
    di = jnp.concatenate([dst.astype(jnp.int32), jnp.zeros((pad,), jnp.int32)])
    xf = x.reshape(R, D)
    acc_rows = -(-(C + 1) // (8 * NS)) * 8 * NS
    z = jnp.zeros((acc_rows, D), jnp.float32)

    sc_call = _make_sc_call(R, D, C, per_tile, N)
    out_sum = sc_call(xf, bi, si, di, z)
    return jnp.concatenate([out_sum.reshape(B, N, D), x], axis=2)


# speedup vs baseline: 1.8922x; 1.5781x over previous
import functools

import jax
import jax.numpy as jnp
from jax import lax
from jax.experimental import pallas as pl
from jax.experimental.pallas import tpu as pltpu
from jax.experimental.pallas import tpu_sc as plsc

NC = 2
NS = 16
BATCH = 128


def _make_sc_call(R, D, C, per_tile, N):
    n_chunks_per_core = (R // C) // NC
    mesh = plsc.VectorSubcoreMesh(
        core_axis_name="c", subcore_axis_name="s",
        num_cores=NC, num_subcores=NS)
    acc_rows = -(-(C + 1) // (8 * NS)) * 8 * NS
    zrows = acc_rows // NS
    wrows = (C // NS) // 8 * 8
    wrem = C - wrows * NS

    @functools.partial(
        pl.kernel,
        out_type=jax.ShapeDtypeStruct((R, D), jnp.float32),
        mesh=mesh,
        scratch_types=[
            pltpu.VMEM_SHARED((acc_rows, 128), jnp.float32),
            pltpu.VMEM((BATCH,), jnp.int32),
            pltpu.VMEM((BATCH,), jnp.int32),
            pltpu.VMEM((BATCH,), jnp.int32),
            pltpu.VMEM((BATCH,), jnp.int32),
            pltpu.VMEM((BATCH,), jnp.int32),
            pltpu.VMEM((BATCH,), jnp.int32),
            pltpu.VMEM((BATCH,), jnp.int32),
            pltpu.VMEM((BATCH, 128), jnp.float32),
            pltpu.VMEM((BATCH, 128), jnp.float32),
            pltpu.SemaphoreType.DMA,
            pltpu.SemaphoreType.DMA,
            pltpu.SemaphoreType.DMA,
        ],
    )
    def sc_call(xf_h, bi_h, si_h, di_h, z_h, out_h,
                acc, biv, siv, div, g0v, g1v, s0v, s1v, st0, st1,
                semi, sem0, sem1):
        c = lax.axis_index("c")
        s = lax.axis_index("s")
        n_batches = per_tile // BATCH

        for p in range(n_chunks_per_core):
            lo = (n_chunks_per_core * c + p) * C
            pltpu.sync_copy(z_h.at[pl.ds(s * zrows, zrows)],
                            acc.at[pl.ds(s * zrows, zrows)])
            plsc.subcore_barrier()

            def body(i, carry):
                base = s * per_tile + i * BATCH
                d_b = pltpu.async_copy(bi_h.at[pl.ds(base, BATCH)], biv, semi)
                d_s = pltpu.async_copy(si_h.at[pl.ds(base, BATCH)], siv, semi)
                d_d = pltpu.async_copy(di_h.at[pl.ds(base, BATCH)], div, semi)
                d_b.wait()
                d_s.wait()
                d_d.wait()
                for j in range(BATCH // 16):
                    sl = pl.ds(j * 16, 16)
                    vb = biv[sl]
                    ls = vb * N + siv[sl]
                    ld = vb * N + div[sl]
                    g0v[sl] = jnp.maximum(ls, 0)
                    g1v[sl] = jnp.maximum(ld, 0)
                    l0 = ld - lo
                    s0v[sl] = jnp.where((l0 < 0) | (l0 >= C), C, l0)
                    l1 = ls - lo
                    s1v[sl] = jnp.where((l1 < 0) | (l1 >= C), C, l1)
                g0 = pltpu.async_copy(xf_h.at[g0v], st0, sem0)
                g1 = pltpu.async_copy(xf_h.at[g1v], st1, sem1)
                g0.wait()
                pltpu.sync_copy(st0, acc.at[s0v], add=True)
                g1.wait()
                pltpu.sync_copy(st1, acc.at[s1v], add=True)
                return carry

            lax.fori_loop(0, n_batches, body, 0)
            plsc.subcore_barrier()
            pltpu.sync_copy(acc.at[pl.ds(s * wrows, wrows)],
                            out_h.at[pl.ds(lo + s * wrows, wrows)])
            if wrem:
                @pl.when(s == 0)
                def _():
                    pltpu.sync_copy(acc.at[pl.ds(NS * wrows, wrem)],
                                    out_h.at[pl.ds(lo + NS * wrows, wrem)])
            plsc.subcore_barrier()

    return sc_call


def kernel(x, batch_idx, src, dst):
    B, N, D = x.shape
    R = B * N
    C = R // 4
    E = batch_idx.shape[0]
    per_tile = -(-E // (NS * BATCH)) * BATCH
    pad = per_tile * NS - E

    bi = jnp.concatenate([batch_idx.astype(jnp.int32),
                          jnp.full((pad,), -1, jnp.int32)])
    si = jnp.concatenate([src.astype(jnp.int32), jnp.zeros((pad,), jnp.int32)])
    di = jnp.concatenate([dst.astype(jnp.int32), jnp.zeros((pad,), jnp.int32)])
    xf = x.reshape(R, D)
    acc_rows = -(-(C + 1) // (8 * NS)) * 8 * NS
    z = jnp.zeros((acc_rows, D), jnp.float32)

    sc_call = _make_sc_call(R, D, C, per_tile, N)
    out_sum = sc_call(xf, bi, si, di, z)
    return jnp.concatenate([out_sum.reshape(B, N, D), x], axis=2)
